# baseline (device time: 263211 ns/iter reference)
import jax
import jax.numpy as jnp
from jax import lax
from jax.experimental import pallas as pl
from jax.experimental.pallas import tpu as pltpu

P = 8
ROWS = ((0, 1368), (1368, 1368), (2736, 1360))
DIMS = ((0, 1, 2), (1, 2, 0), (2, 0, 1))


def kernel(x, w_mat):
    m, k_local = x.shape
    _, n = w_mat.shape

    def body(x_ref, w_ref, out_ref, rbuf1, rbuf2, ssems, rsems):
        d = lax.axis_index("i")
        zb = d // 4
        q = d % 4
        xb = ((q == 1) | (q == 2)).astype(jnp.int32)
        yb = q // 2
        bits = (xb, yb, zb)
        partner = (
            zb * 4 + q + 1 - 2 * (q % 2),
            zb * 4 + 3 - q,
            (d + 4) % P,
        )

        barrier_sem = pltpu.get_barrier_semaphore()
        for dim in range(3):
            pl.semaphore_signal(
                barrier_sem, inc=1,
                device_id=(partner[dim],), device_id_type=pl.DeviceIdType.MESH,
            )
        pl.semaphore_wait(barrier_sem, 3)

        O = []
        for p in range(3):
            b0 = bits[DIMS[p][0]]
            b1 = bits[DIMS[p][1]]
            b2 = bits[DIMS[p][2]]
            away0 = (1 - b0) * 1024
            o1 = b0 * 1024
            O.append(dict(
                away0=away0,
                o1=o1,
                sub1=away0 + (1 - b1) * 512,
                sub2=away0 + b1 * 512,
                away1=o1 + (1 - b1) * 512,
                o2=o1 + b1 * 512,
                away2=o1 + b1 * 512 + (1 - b2) * 256,
                o3=o1 + b1 * 512 + b2 * 256,
                c2=o1 + b1 * 512 + (1 - b2) * 256,
                h=o1 + (1 - b1) * 512,
            ))

        def rows(p):
            return pl.ds(ROWS[p][0], ROWS[p][1])

        def oslc(p, off, width):
            return out_ref.at[rows(p), pl.ds(off, width)]

        def gemm(p, coff, width):
            return jnp.dot(
                x_ref[rows(p), :],
                w_ref[:, pl.ds(coff, width)],
                preferred_element_type=jnp.float32,
            )

        def ex(p, idx, dim, src, dst):
            c = pltpu.make_async_remote_copy(
                src_ref=src, dst_ref=dst,
                send_sem=ssems.at[p, idx], recv_sem=rsems.at[p, idx],
                device_id=(partner[dim],),
                device_id_type=pl.DeviceIdType.MESH,
            )
            c.start()
            return c

        E = {}

        for p in range(3):
            out_ref[rows(p), pl.ds(O[p]["sub1"], 512)] = gemm(p, O[p]["sub1"], 512)
        for p in range(3):
            s = oslc(p, O[p]["sub1"], 512)
            E[p, "0a"] = ex(p, 0, DIMS[p][0], s, s)
        for p in range(3):
            out_ref[rows(p), pl.ds(O[p]["sub2"], 512)] = gemm(p, O[p]["sub2"], 512)
        for p in range(3):
            s = oslc(p, O[p]["sub2"], 512)
            E[p, "0b"] = ex(p, 1, DIMS[p][0], s, s)

        for p in range(3):
            E[p, "0a"].wait()
            a1 = O[p]["away1"]
            out_ref[rows(p), pl.ds(a1, 512)] = (
                out_ref[rows(p), pl.ds(a1, 512)] + gemm(p, a1, 512)
            )
            E[p, "1"] = ex(p, 2, DIMS[p][1], oslc(p, a1, 512),
                           rbuf1.at[p, pl.ds(0, ROWS[p][1]), :])
        for p in range(3):
            E[p, "0b"].wait()
            o2 = O[p]["o2"]
            out_ref[rows(p), pl.ds(o2, 512)] = (
                out_ref[rows(p), pl.ds(o2, 512)] + gemm(p, o2, 512)
            )

        for p in range(3):
            E[p, "1"].wait()
            a2, o2 = O[p]["away2"], O[p]["o2"]
            rn = ROWS[p][1]
            out_ref[rows(p), pl.ds(a2, 256)] = (
                out_ref[rows(p), pl.ds(a2, 256)]
                + rbuf1[p, pl.ds(0, rn), pl.ds(a2 - o2, 256)]
            )
            E[p, "2"] = ex(p, 3, DIMS[p][2], oslc(p, a2, 256),
                           rbuf2.at[p, pl.ds(0, rn), :])
        for p in range(3):
            o2, o3 = O[p]["o2"], O[p]["o3"]
            rn = ROWS[p][1]
            out_ref[rows(p), pl.ds(o3, 256)] = (
                out_ref[rows(p), pl.ds(o3, 256)]
                + rbuf1[p, pl.ds(0, rn), pl.ds(o3 - o2, 256)]
            )
        for p in range(3):
            E[p, "2"].wait()
            o3 = O[p]["o3"]
            rn = ROWS[p][1]
            out_ref[rows(p), pl.ds(o3, 256)] = jnp.maximum(
                out_ref[rows(p), pl.ds(o3, 256)] + rbuf2[p, pl.ds(0, rn), :],
                0.0,
            )

        for p in range(3):
            s = oslc(p, O[p]["o3"], 256)
            E[p, "A1"] = ex(p, 4, DIMS[p][2], s, s)
            E[p, "A2a"] = ex(p, 5, DIMS[p][1], s, s)
            E[p, "A3a"] = ex(p, 6, DIMS[p][0], s, s)
        for p in range(3):
            E[p, "A1"].wait()
            s = oslc(p, O[p]["c2"], 256)
            E[p, "A2b"] = ex(p, 7, DIMS[p][1], s, s)
            E[p, "A3b"] = ex(p, 8, DIMS[p][0], s, s)
        for p in range(3):
            E[p, "A2a"].wait()
            E[p, "A2b"].wait()
            s = oslc(p, O[p]["h"], 512)
            E[p, "A3c"] = ex(p, 9, DIMS[p][0], s, s)
        for p in range(3):
            E[p, "A3a"].wait()
            E[p, "A3b"].wait()
            E[p, "A3c"].wait()

    return pl.pallas_call(
        body,
        out_shape=jax.ShapeDtypeStruct((m, n), jnp.float32),
        in_specs=[
            pl.BlockSpec(memory_space=pltpu.VMEM),
            pl.BlockSpec(memory_space=pltpu.VMEM),
        ],
        out_specs=pl.BlockSpec(memory_space=pltpu.VMEM),
        scratch_shapes=[
            pltpu.VMEM((3, 1368, 512), jnp.float32),
            pltpu.VMEM((3, 1368, 256), jnp.float32),
            pltpu.SemaphoreType.DMA((3, 10)),
            pltpu.SemaphoreType.DMA((3, 10)),
        ],
        compiler_params=pltpu.CompilerParams(
            collective_id=0,
            vmem_limit_bytes=63 * 1024 * 1024,
        ),
    )(x, w_mat)


# device time: 262227 ns/iter; 1.0038x vs baseline; 1.0038x over previous
import jax
import jax.numpy as jnp
from jax import lax
from jax.experimental import pallas as pl
from jax.experimental.pallas import tpu as pltpu

P = 8
ROWS = ((0, 1368), (1368, 1368), (2736, 1360))
DIMS = ((0, 1, 2), (1, 2, 0), (2, 0, 1))


def kernel(x, w_mat):
    m, k_local = x.shape
    _, n = w_mat.shape

    def body(x_ref, w_ref, out_ref, rbuf1, rbuf2, ssems, rsems):
        d = lax.axis_index("i")
        zb = d // 4
        q = d % 4
        xb = ((q == 1) | (q == 2)).astype(jnp.int32)
        yb = q // 2
        bits = (xb, yb, zb)
        partner = (
            zb * 4 + q + 1 - 2 * (q % 2),
            zb * 4 + 3 - q,
            (d + 4) % P,
        )

        barrier_sem = pltpu.get_barrier_semaphore()
        for dim in range(3):
            pl.semaphore_signal(
                barrier_sem, inc=1,
                device_id=(partner[dim],), device_id_type=pl.DeviceIdType.MESH,
            )
        pl.semaphore_wait(barrier_sem, 3)

        O = []
        for p in range(3):
            b0 = bits[DIMS[p][0]]
            b1 = bits[DIMS[p][1]]
            b2 = bits[DIMS[p][2]]
            away0 = (1 - b0) * 1024
            o1 = b0 * 1024
            O.append(dict(
                away0=away0,
                o1=o1,
                sub1=away0 + (1 - b1) * 512,
                sub2=away0 + b1 * 512,
                away1=o1 + (1 - b1) * 512,
                o2=o1 + b1 * 512,
                away2=o1 + b1 * 512 + (1 - b2) * 256,
                o3=o1 + b1 * 512 + b2 * 256,
                c2=o1 + b1 * 512 + (1 - b2) * 256,
                h=o1 + (1 - b1) * 512,
            ))

        def rows(p):
            return pl.ds(ROWS[p][0], ROWS[p][1])

        def oslc(p, off, width):
            return out_ref.at[rows(p), pl.ds(off, width)]

        def gemm(p, coff, width):
            return jnp.dot(
                x_ref[rows(p), :],
                w_ref[:, pl.ds(coff, width)],
                preferred_element_type=jnp.float32,
            )

        def ex(p, idx, dim, src, dst):
            c = pltpu.make_async_remote_copy(
                src_ref=src, dst_ref=dst,
                send_sem=ssems.at[p, idx], recv_sem=rsems.at[p, idx],
                device_id=(partner[dim],),
                device_id_type=pl.DeviceIdType.MESH,
            )
            c.start()
            return c

        E = {}

        for p in range(3):
            out_ref[rows(p), pl.ds(O[p]["sub1"], 512)] = gemm(p, O[p]["sub1"], 512)
            s = oslc(p, O[p]["sub1"], 512)
            E[p, "0a"] = ex(p, 0, DIMS[p][0], s, s)
        for p in range(3):
            out_ref[rows(p), pl.ds(O[p]["sub2"], 512)] = gemm(p, O[p]["sub2"], 512)
            s = oslc(p, O[p]["sub2"], 512)
            E[p, "0b"] = ex(p, 1, DIMS[p][0], s, s)

        t_away1 = {p: gemm(p, O[p]["away1"], 512) for p in (0, 1)}

        for p in range(3):
            E[p, "0a"].wait()
            a1 = O[p]["away1"]
            t = t_away1[p] if p in t_away1 else gemm(p, a1, 512)
            out_ref[rows(p), pl.ds(a1, 512)] = (
                out_ref[rows(p), pl.ds(a1, 512)] + t
            )
            E[p, "1"] = ex(p, 2, DIMS[p][1], oslc(p, a1, 512),
                           rbuf1.at[p, pl.ds(0, ROWS[p][1]), :])
        for p in range(3):
            E[p, "0b"].wait()
            o2 = O[p]["o2"]
            out_ref[rows(p), pl.ds(o2, 512)] = (
                out_ref[rows(p), pl.ds(o2, 512)] + gemm(p, o2, 512)
            )

        for p in range(3):
            E[p, "1"].wait()
            a2, o2 = O[p]["away2"], O[p]["o2"]
            rn = ROWS[p][1]
            out_ref[rows(p), pl.ds(a2, 256)] = (
                out_ref[rows(p), pl.ds(a2, 256)]
                + rbuf1[p, pl.ds(0, rn), pl.ds(a2 - o2, 256)]
            )
            E[p, "2"] = ex(p, 3, DIMS[p][2], oslc(p, a2, 256),
                           rbuf2.at[p, pl.ds(0, rn), :])
        for p in range(3):
            o2, o3 = O[p]["o2"], O[p]["o3"]
            rn = ROWS[p][1]
            out_ref[rows(p), pl.ds(o3, 256)] = (
                out_ref[rows(p), pl.ds(o3, 256)]
                + rbuf1[p, pl.ds(0, rn), pl.ds(o3 - o2, 256)]
            )
        for p in range(3):
            E[p, "2"].wait()
            o3 = O[p]["o3"]
            rn = ROWS[p][1]
            out_ref[rows(p), pl.ds(o3, 256)] = jnp.maximum(
                out_ref[rows(p), pl.ds(o3, 256)] + rbuf2[p, pl.ds(0, rn), :],
                0.0,
            )
            s = oslc(p, O[p]["o3"], 256)
            E[p, "A1"] = ex(p, 4, DIMS[p][2], s, s)
            E[p, "A2a"] = ex(p, 5, DIMS[p][1], s, s)
            E[p, "A3a"] = ex(p, 6, DIMS[p][0], s, s)
        for p in range(3):
            E[p, "A1"].wait()
            s = oslc(p, O[p]["c2"], 256)
            E[p, "A2b"] = ex(p, 7, DIMS[p][1], s, s)
            E[p, "A3b"] = ex(p, 8, DIMS[p][0], s, s)
        for p in range(3):
            E[p, "A2a"].wait()
            E[p, "A2b"].wait()
            s = oslc(p, O[p]["h"], 512)
            E[p, "A3c"] = ex(p, 9, DIMS[p][0], s, s)
        for p in range(3):
            E[p, "A3a"].wait()
            E[p, "A3b"].wait()
            E[p, "A3c"].wait()

    return pl.pallas_call(
        body,
        out_shape=jax.ShapeDtypeStruct((m, n), jnp.float32),
        in_specs=[
            pl.BlockSpec(memory_space=pltpu.VMEM),
            pl.BlockSpec(memory_space=pltpu.VMEM),
        ],
        out_specs=pl.BlockSpec(memory_space=pltpu.VMEM),
        scratch_shapes=[
            pltpu.VMEM((3, 1368, 512), jnp.float32),
            pltpu.VMEM((3, 1368, 256), jnp.float32),
            pltpu.SemaphoreType.DMA((3, 10)),
            pltpu.SemaphoreType.DMA((3, 10)),
        ],
        compiler_params=pltpu.CompilerParams(
            collective_id=0,
            vmem_limit_bytes=63 * 1024 * 1024,
        ),
    )(x, w_mat)


# device time: 252687 ns/iter; 1.0416x vs baseline; 1.0378x over previous
import jax
import jax.numpy as jnp
from jax import lax
from jax.experimental import pallas as pl
from jax.experimental.pallas import tpu as pltpu

P = 8
ROWS = ((0, 1368), (1368, 1368), (2736, 1360))
DIMS = ((0, 1, 2), (1, 2, 0), (2, 0, 1))


def kernel(x, w_mat):
    m, k_local = x.shape
    _, n = w_mat.shape

    def body(x_ref, w_ref, out_ref, wb, rbuf1, rbuf2, ssems, rsems, lsems):
        d = lax.axis_index("i")
        zb = d // 4
        q = d % 4
        xb = ((q == 1) | (q == 2)).astype(jnp.int32)
        yb = q // 2
        bits = (xb, yb, zb)
        partner = (
            zb * 4 + q + 1 - 2 * (q % 2),
            zb * 4 + 3 - q,
            (d + 4) % P,
        )

        barrier_sem = pltpu.get_barrier_semaphore()
        for dim in range(3):
            pl.semaphore_signal(
                barrier_sem, inc=1,
                device_id=(partner[dim],), device_id_type=pl.DeviceIdType.MESH,
            )
        pl.semaphore_wait(barrier_sem, 3)

        O = []
        for p in range(3):
            b0 = bits[DIMS[p][0]]
            b1 = bits[DIMS[p][1]]
            b2 = bits[DIMS[p][2]]
            away0 = (1 - b0) * 1024
            o1 = b0 * 1024
            O.append(dict(
                away0=away0,
                o1=o1,
                sub1=away0 + (1 - b1) * 512,
                sub2=away0 + b1 * 512,
                away1=o1 + (1 - b1) * 512,
                o2=o1 + b1 * 512,
                away2=o1 + b1 * 512 + (1 - b2) * 256,
                o3=o1 + b1 * 512 + b2 * 256,
                c2=o1 + b1 * 512 + (1 - b2) * 256,
                h=o1 + (1 - b1) * 512,
            ))

        def wslc(p, off, width):
            return wb.at[p, pl.ds(0, ROWS[p][1]), pl.ds(off, width)]

        def oslc(p, off, width):
            return out_ref.at[pl.ds(ROWS[p][0], ROWS[p][1]), pl.ds(off, width)]

        def gemm(p, coff, width):
            return jnp.dot(
                x_ref[pl.ds(ROWS[p][0], ROWS[p][1]), :],
                w_ref[:, pl.ds(coff, width)],
                preferred_element_type=jnp.float32,
            )

        def ex(p, idx, dim, src, dst):
            c = pltpu.make_async_remote_copy(
                src_ref=src, dst_ref=dst,
                send_sem=ssems.at[p, idx], recv_sem=rsems.at[p, idx],
                device_id=(partner[dim],),
                device_id_type=pl.DeviceIdType.MESH,
            )
            c.start()
            return c

        E = {}

        for p in range(3):
            rn = ROWS[p][1]
            wb[p, pl.ds(0, rn), pl.ds(O[p]["sub1"], 512)] = gemm(p, O[p]["sub1"], 512)
            s = wslc(p, O[p]["sub1"], 512)
            E[p, "0a"] = ex(p, 0, DIMS[p][0], s, s)
        for p in range(3):
            rn = ROWS[p][1]
            wb[p, pl.ds(0, rn), pl.ds(O[p]["sub2"], 512)] = gemm(p, O[p]["sub2"], 512)
            s = wslc(p, O[p]["sub2"], 512)
            E[p, "0b"] = ex(p, 1, DIMS[p][0], s, s)

        for p in range(3):
            rn = ROWS[p][1]
            E[p, "0a"].wait()
            a1 = O[p]["away1"]
            wb[p, pl.ds(0, rn), pl.ds(a1, 512)] = (
                wb[p, pl.ds(0, rn), pl.ds(a1, 512)] + gemm(p, a1, 512)
            )
            E[p, "1"] = ex(p, 2, DIMS[p][1], wslc(p, a1, 512),
                           rbuf1.at[p, pl.ds(0, rn), :])
        for p in range(3):
            rn = ROWS[p][1]
            E[p, "0b"].wait()
            o2 = O[p]["o2"]
            wb[p, pl.ds(0, rn), pl.ds(o2, 512)] = (
                wb[p, pl.ds(0, rn), pl.ds(o2, 512)] + gemm(p, o2, 512)
            )

        for p in range(3):
            rn = ROWS[p][1]
            E[p, "1"].wait()
            a2, o2 = O[p]["away2"], O[p]["o2"]
            wb[p, pl.ds(0, rn), pl.ds(a2, 256)] = (
                wb[p, pl.ds(0, rn), pl.ds(a2, 256)]
                + rbuf1[p, pl.ds(0, rn), pl.ds(a2 - o2, 256)]
            )
            E[p, "2"] = ex(p, 3, DIMS[p][2], wslc(p, a2, 256),
                           rbuf2.at[p, pl.ds(0, rn), :])
        for p in range(3):
            rn = ROWS[p][1]
            o2, o3 = O[p]["o2"], O[p]["o3"]
            wb[p, pl.ds(0, rn), pl.ds(o3, 256)] = (
                wb[p, pl.ds(0, rn), pl.ds(o3, 256)]
                + rbuf1[p, pl.ds(0, rn), pl.ds(o3 - o2, 256)]
            )

        CP = {}

        def out_cp(p, idx, off, width):
            c = pltpu.make_async_copy(
                wslc(p, off, width), oslc(p, off, width), lsems.at[p, idx]
            )
            c.start()
            return c

        for p in range(3):
            rn = ROWS[p][1]
            E[p, "2"].wait()
            o3 = O[p]["o3"]
            wb[p, pl.ds(0, rn), pl.ds(o3, 256)] = jnp.maximum(
                wb[p, pl.ds(0, rn), pl.ds(o3, 256)] + rbuf2[p, pl.ds(0, rn), :],
                0.0,
            )
            s = wslc(p, O[p]["o3"], 256)
            E[p, "A1"] = ex(p, 4, DIMS[p][2], s, s)
            E[p, "A2a"] = ex(p, 5, DIMS[p][1], s, s)
            E[p, "A3a"] = ex(p, 6, DIMS[p][0], s, s)
        for p in range(3):
            E[p, "A1"].wait()
            CP[p, 0] = out_cp(p, 0, O[p]["o2"], 512)
            s = wslc(p, O[p]["c2"], 256)
            E[p, "A2b"] = ex(p, 7, DIMS[p][1], s, s)
            E[p, "A3b"] = ex(p, 8, DIMS[p][0], s, s)
        for p in range(3):
            E[p, "A2a"].wait()
            E[p, "A2b"].wait()
            CP[p, 1] = out_cp(p, 1, O[p]["h"], 512)
            E[p, "A3c"] = ex(p, 9, DIMS[p][0], wslc(p, O[p]["h"], 512),
                             wslc(p, O[p]["h"], 512))
        b1away = [O[p]["away0"] + bits[DIMS[p][1]] * 512 for p in range(3)]
        for p in range(3):
            E[p, "A3a"].wait()
            E[p, "A3b"].wait()
            CP[p, 2] = out_cp(p, 2, b1away[p], 512)
        for p in range(3):
            E[p, "A3c"].wait()
            CP[p, 3] = out_cp(p, 3, O[p]["away0"] + (1 - bits[DIMS[p][1]]) * 512,
                              512)
        for p in range(3):
            for i in range(4):
                CP[p, i].wait()

    return pl.pallas_call(
        body,
        out_shape=jax.ShapeDtypeStruct((m, n), jnp.float32),
        in_specs=[
            pl.BlockSpec(memory_space=pltpu.VMEM),
            pl.BlockSpec(memory_space=pltpu.VMEM),
        ],
        out_specs=pl.BlockSpec(memory_space=pl.ANY),
        scratch_shapes=[
            pltpu.VMEM((3, 1368, 2048), jnp.float32),
            pltpu.VMEM((3, 1368, 512), jnp.float32),
            pltpu.VMEM((3, 1368, 256), jnp.float32),
            pltpu.SemaphoreType.DMA((3, 10)),
            pltpu.SemaphoreType.DMA((3, 10)),
            pltpu.SemaphoreType.DMA((3, 4)),
        ],
        compiler_params=pltpu.CompilerParams(
            collective_id=0,
            vmem_limit_bytes=63 * 1024 * 1024,
        ),
    )(x, w_mat)


# device time: 252480 ns/iter; 1.0425x vs baseline; 1.0008x over previous
import jax
import jax.numpy as jnp
from jax import lax
from jax.experimental import pallas as pl
from jax.experimental.pallas import tpu as pltpu

P = 8
ROWS = ((0, 1368), (1368, 1368), (2736, 1360))
DIMS = ((0, 1, 2), (1, 2, 0), (2, 0, 1))


def kernel(x, w_mat):
    m, k_local = x.shape
    _, n = w_mat.shape

    def body(x_ref, w_ref, out_ref, wb, rbuf1, rbuf2, ssems, rsems, lsems):
        d = lax.axis_index("i")
        zb = d // 4
        q = d % 4
        xb = ((q == 1) | (q == 2)).astype(jnp.int32)
        yb = q // 2
        bits = (xb, yb, zb)
        partner = (
            zb * 4 + q + 1 - 2 * (q % 2),
            zb * 4 + 3 - q,
            (d + 4) % P,
        )

        barrier_sem = pltpu.get_barrier_semaphore()
        for dim in range(3):
            pl.semaphore_signal(
                barrier_sem, inc=1,
                device_id=(partner[dim],), device_id_type=pl.DeviceIdType.MESH,
            )
        pl.semaphore_wait(barrier_sem, 3)

        O = []
        for p in range(3):
            b0 = bits[DIMS[p][0]]
            b1 = bits[DIMS[p][1]]
            b2 = bits[DIMS[p][2]]
            away0 = (1 - b0) * 1024
            o1 = b0 * 1024
            O.append(dict(
                away0=away0,
                o1=o1,
                sub1=away0 + (1 - b1) * 512,
                sub2=away0 + b1 * 512,
                away1=o1 + (1 - b1) * 512,
                o2=o1 + b1 * 512,
                away2=o1 + b1 * 512 + (1 - b2) * 256,
                o3=o1 + b1 * 512 + b2 * 256,
                c2=o1 + b1 * 512 + (1 - b2) * 256,
                h=o1 + (1 - b1) * 512,
            ))

        def wslc(p, off, width):
            return wb.at[p, pl.ds(0, ROWS[p][1]), pl.ds(off, width)]

        def oslc(p, off, width):
            return out_ref.at[pl.ds(ROWS[p][0], ROWS[p][1]), pl.ds(off, width)]

        def gemm(p, coff, width):
            return jnp.dot(
                x_ref[pl.ds(ROWS[p][0], ROWS[p][1]), :],
                w_ref[:, pl.ds(coff, width)],
                preferred_element_type=jnp.float32,
            )

        def ex(p, idx, dim, src, dst):
            c = pltpu.make_async_remote_copy(
                src_ref=src, dst_ref=dst,
                send_sem=ssems.at[p, idx], recv_sem=rsems.at[p, idx],
                device_id=(partner[dim],),
                device_id_type=pl.DeviceIdType.MESH,
            )
            c.start()
            return c

        E = {}

        for p in range(3):
            rn = ROWS[p][1]
            wb[p, pl.ds(0, rn), pl.ds(O[p]["sub1"], 512)] = gemm(p, O[p]["sub1"], 512)
            s = wslc(p, O[p]["sub1"], 512)
            E[p, "0a"] = ex(p, 0, DIMS[p][0], s, s)
        for p in range(3):
            rn = ROWS[p][1]
            wb[p, pl.ds(0, rn), pl.ds(O[p]["sub2"], 512)] = gemm(p, O[p]["sub2"], 512)
            s = wslc(p, O[p]["sub2"], 512)
            E[p, "0b"] = ex(p, 1, DIMS[p][0], s, s)

        for p in range(3):
            rn = ROWS[p][1]
            E[p, "0a"].wait()
            a1 = O[p]["away1"]
            wb[p, pl.ds(0, rn), pl.ds(a1, 512)] = (
                wb[p, pl.ds(0, rn), pl.ds(a1, 512)] + gemm(p, a1, 512)
            )
            E[p, "1"] = ex(p, 2, DIMS[p][1], wslc(p, a1, 512),
                           rbuf1.at[p, pl.ds(0, rn), :])
        for p in range(3):
            rn = ROWS[p][1]
            E[p, "0b"].wait()
            o2, o3, a2 = O[p]["o2"], O[p]["o3"], O[p]["away2"]
            wb[p, pl.ds(0, rn), pl.ds(o2, 512)] = (
                wb[p, pl.ds(0, rn), pl.ds(o2, 512)] + gemm(p, o2, 512)
            )
            E[p, "1"].wait()
            wb[p, pl.ds(0, rn), pl.ds(a2, 256)] = (
                wb[p, pl.ds(0, rn), pl.ds(a2, 256)]
                + rbuf1[p, pl.ds(0, rn), pl.ds(a2 - o2, 256)]
            )
            E[p, "2"] = ex(p, 3, DIMS[p][2], wslc(p, a2, 256),
                           rbuf2.at[p, pl.ds(0, rn), :])
            wb[p, pl.ds(0, rn), pl.ds(o3, 256)] = (
                wb[p, pl.ds(0, rn), pl.ds(o3, 256)]
                + rbuf1[p, pl.ds(0, rn), pl.ds(o3 - o2, 256)]
            )

        CP = {}

        def out_cp(p, idx, off, width):
            c = pltpu.make_async_copy(
                wslc(p, off, width), oslc(p, off, width), lsems.at[p, idx]
            )
            c.start()
            return c

        for p in range(3):
            rn = ROWS[p][1]
            E[p, "2"].wait()
            o3 = O[p]["o3"]
            wb[p, pl.ds(0, rn), pl.ds(o3, 256)] = jnp.maximum(
                wb[p, pl.ds(0, rn), pl.ds(o3, 256)] + rbuf2[p, pl.ds(0, rn), :],
                0.0,
            )
            s = wslc(p, O[p]["o3"], 256)
            E[p, "A1"] = ex(p, 4, DIMS[p][2], s, s)
            E[p, "A2a"] = ex(p, 5, DIMS[p][1], s, s)
            E[p, "A3a"] = ex(p, 6, DIMS[p][0], s, s)
        for p in range(3):
            E[p, "A1"].wait()
            CP[p, 0] = out_cp(p, 0, O[p]["o2"], 512)
            s = wslc(p, O[p]["c2"], 256)
            E[p, "A2b"] = ex(p, 7, DIMS[p][1], s, s)
            E[p, "A3b"] = ex(p, 8, DIMS[p][0], s, s)
        for p in range(3):
            E[p, "A2a"].wait()
            E[p, "A2b"].wait()
            CP[p, 1] = out_cp(p, 1, O[p]["h"], 512)
            E[p, "A3c"] = ex(p, 9, DIMS[p][0], wslc(p, O[p]["h"], 512),
                             wslc(p, O[p]["h"], 512))
        b1away = [O[p]["away0"] + bits[DIMS[p][1]] * 512 for p in range(3)]
        for p in range(3):
            E[p, "A3a"].wait()
            E[p, "A3b"].wait()
            CP[p, 2] = out_cp(p, 2, b1away[p], 512)
        for p in range(3):
            E[p, "A3c"].wait()
            CP[p, 3] = out_cp(p, 3, O[p]["away0"] + (1 - bits[DIMS[p][1]]) * 512,
                              512)
        for p in range(3):
            for i in range(4):
                CP[p, i].wait()

    return pl.pallas_call(
        body,
        out_shape=jax.ShapeDtypeStruct((m, n), jnp.float32),
        in_specs=[
            pl.BlockSpec(memory_space=pltpu.VMEM),
            pl.BlockSpec(memory_space=pltpu.VMEM),
        ],
        out_specs=pl.BlockSpec(memory_space=pl.ANY),
        scratch_shapes=[
            pltpu.VMEM((3, 1368, 2048), jnp.float32),
            pltpu.VMEM((3, 1368, 512), jnp.float32),
            pltpu.VMEM((3, 1368, 256), jnp.float32),
            pltpu.SemaphoreType.DMA((3, 10)),
            pltpu.SemaphoreType.DMA((3, 10)),
            pltpu.SemaphoreType.DMA((3, 4)),
        ],
        compiler_params=pltpu.CompilerParams(
            collective_id=0,
            vmem_limit_bytes=63 * 1024 * 1024,
        ),
    )(x, w_mat)
